# baseline (speedup 1.0000x reference)
import functools

import jax
import jax.numpy as jnp
from jax import lax
from jax.experimental import pallas as pl
from jax.experimental.pallas import tpu as pltpu
from jax.experimental.pallas import tpu_sc as plsc

V = 10000
D = 128
B = 320000
NC, NS = 2, 16
NW = NC * NS
BPW = B // NW
CH = 80
NCH = BPW // CH
KA = 4
SLAB = 400
NSLAB = V // SLAB

_mesh = plsc.VectorSubcoreMesh(core_axis_name="c", subcore_axis_name="s")


@functools.partial(
    pl.kernel,
    mesh=_mesh,
    out_type=jax.ShapeDtypeStruct((B, D), jnp.float32),
    scratch_types=[
        pltpu.VMEM((NCH, CH), jnp.int32),
        pltpu.VMEM((2, CH, D), jnp.float32),
        pltpu.VMEM_SHARED((V, D), jnp.float32),
        pltpu.SemaphoreType.DMA,
        pltpu.SemaphoreType.DMA,
        pltpu.SemaphoreType.DMA,
        pltpu.SemaphoreType.DMA,
        pltpu.SemaphoreType.DMA,
    ],
)
def _gather_kernel(table_hbm, idx_hbm, out_hbm, idx_v, bufs, table_sp,
                   g0, g1, o0, o1, ssem):
    sid = lax.axis_index("s")
    wid = sid * NC + lax.axis_index("c")
    base = wid * BPW
    gsem = (g0, g1)
    osem = (o0, o1)

    def slab_copy(c):
        return pltpu.make_async_copy(
            table_hbm.at[pl.ds(c * SLAB, SLAB)],
            table_sp.at[pl.ds(c * SLAB, SLAB)],
            ssem,
        )

    for c in range(NSLAB):
        @pl.when(sid == c % NS)
        def _():
            slab_copy(c).start()

    pltpu.sync_copy(idx_hbm.at[wid], idx_v)

    def fire_chunk(g, p, src):
        pltpu.async_copy(src.at[idx_v.at[g]], bufs.at[p], gsem[p])

    def drain_chunk(p):
        pltpu.make_async_copy(
            table_hbm.at[pl.ds(0, CH)], bufs.at[p], gsem[p]
        ).wait()

    def out_copy(g, p):
        pltpu.async_copy(bufs.at[p], out_hbm.at[pl.ds(base + g * CH, CH)], osem[p])

    def drain_out(p):
        pltpu.make_async_copy(
            bufs.at[p], out_hbm.at[pl.ds(base, CH)], osem[p]
        ).wait()

    def make_body(src, fire_bound):
        def body(i, _):
            for p in (0, 1):
                g = 2 * i + p

                @pl.when(g >= 1)
                def _():
                    drain_out(1 - p)

                @pl.when(g + 1 < fire_bound)
                def _():
                    fire_chunk(g + 1, 1 - p, src)

                @pl.when(g < NCH)
                def _():
                    drain_chunk(p)

                @pl.when(g < NCH)
                def _():
                    out_copy(g, p)
            return 0
        return body

    fire_chunk(0, 0, table_hbm)
    lax.fori_loop(0, KA // 2, make_body(table_hbm, KA), 0)

    for c in range(NSLAB):
        @pl.when(sid == c % NS)
        def _():
            slab_copy(c).wait()
    plsc.subcore_barrier()

    fire_chunk(KA, 0, table_sp)
    lax.fori_loop(KA // 2, (NCH + 2) // 2, make_body(table_sp, NCH), 0)


def kernel(parent_features, child_to_parent_idx):
    idx3d = child_to_parent_idx.astype(jnp.int32).reshape(NW, NCH, CH)
    return _gather_kernel(parent_features, idx3d)


# speedup vs baseline: 1.0075x; 1.0075x over previous
import functools

import jax
import jax.numpy as jnp
from jax import lax
from jax.experimental import pallas as pl
from jax.experimental.pallas import tpu as pltpu
from jax.experimental.pallas import tpu_sc as plsc

V = 10000
D = 128
B = 320000
NC, NS = 2, 16
NW = NC * NS
BPW = B // NW
CH = 80
NCH = BPW // CH
KA = 2
SLAB = 400
NSLAB = V // SLAB

_mesh = plsc.VectorSubcoreMesh(core_axis_name="c", subcore_axis_name="s")


@functools.partial(
    pl.kernel,
    mesh=_mesh,
    out_type=jax.ShapeDtypeStruct((B, D), jnp.float32),
    scratch_types=[
        pltpu.VMEM((NCH, CH), jnp.int32),
        pltpu.VMEM((2, CH, D), jnp.float32),
        pltpu.VMEM_SHARED((V, D), jnp.float32),
        pltpu.SemaphoreType.DMA,
        pltpu.SemaphoreType.DMA,
        pltpu.SemaphoreType.DMA,
        pltpu.SemaphoreType.DMA,
        pltpu.SemaphoreType.DMA,
    ],
)
def _gather_kernel(table_hbm, idx_hbm, out_hbm, idx_v, bufs, table_sp,
                   g0, g1, o0, o1, ssem):
    sid = lax.axis_index("s")
    wid = sid * NC + lax.axis_index("c")
    base = wid * BPW
    gsem = (g0, g1)
    osem = (o0, o1)

    def slab_copy(c):
        return pltpu.make_async_copy(
            table_hbm.at[pl.ds(c * SLAB, SLAB)],
            table_sp.at[pl.ds(c * SLAB, SLAB)],
            ssem,
        )

    for c in range(NSLAB):
        @pl.when(sid == c % NS)
        def _():
            slab_copy(c).start()

    pltpu.sync_copy(idx_hbm.at[wid], idx_v)

    def fire_chunk(g, p, src):
        pltpu.async_copy(src.at[idx_v.at[g]], bufs.at[p], gsem[p])

    def drain_chunk(p):
        pltpu.make_async_copy(
            table_hbm.at[pl.ds(0, CH)], bufs.at[p], gsem[p]
        ).wait()

    def out_copy(g, p):
        pltpu.async_copy(bufs.at[p], out_hbm.at[pl.ds(base + g * CH, CH)], osem[p])

    def drain_out(p):
        pltpu.make_async_copy(
            bufs.at[p], out_hbm.at[pl.ds(base, CH)], osem[p]
        ).wait()

    def make_body(src, fire_bound):
        def body(i, _):
            for p in (0, 1):
                g = 2 * i + p

                @pl.when(g >= 1)
                def _():
                    drain_out(1 - p)

                @pl.when(g + 1 < fire_bound)
                def _():
                    fire_chunk(g + 1, 1 - p, src)

                @pl.when(g < NCH)
                def _():
                    drain_chunk(p)

                @pl.when(g < NCH)
                def _():
                    out_copy(g, p)
            return 0
        return body

    fire_chunk(0, 0, table_hbm)
    lax.fori_loop(0, KA // 2, make_body(table_hbm, KA), 0)

    for c in range(NSLAB):
        @pl.when(sid == c % NS)
        def _():
            slab_copy(c).wait()
    plsc.subcore_barrier()

    fire_chunk(KA, 0, table_sp)
    lax.fori_loop(KA // 2, (NCH + 2) // 2, make_body(table_sp, NCH), 0)


def kernel(parent_features, child_to_parent_idx):
    idx3d = child_to_parent_idx.astype(jnp.int32).reshape(NW, NCH, CH)
    return _gather_kernel(parent_features, idx3d)
